# Initial kernel scaffold; baseline (speedup 1.0000x reference)
import jax
import jax.numpy as jnp
from jax.experimental import pallas as pl


def kernel(z_e, codebook):
    raise NotImplementedError("write your pallas kernel here")


# speedup vs baseline: 2.9858x; 2.9858x over previous
import functools

import jax
import jax.numpy as jnp
from jax import lax
from jax.experimental import pallas as pl
from jax.experimental.pallas import tpu as pltpu

_B = 16384
_K = 1024
_D = 64
_RB = 512
_NB = _B // _RB


def _vq_block(z_ref, cb_ref, zq_soft_ref, zq_hard_ref, idx_ref, w_ref, loss_ref):
    i = pl.program_id(0)
    z = z_ref[...]
    cb = cb_ref[...]

    zsq = jnp.sum(z * z, axis=1, keepdims=True)
    csq = jnp.sum(cb * cb, axis=1)[None, :]
    mm = lax.dot_general(z, cb, (((1,), (1,)), ((), ())),
                         preferred_element_type=jnp.float32)
    dists = zsq - 2.0 * mm + csq

    min_d = jnp.min(dists, axis=1, keepdims=True)
    iota_k = lax.broadcasted_iota(jnp.int32, (_RB, _K), 1)
    idx = jnp.min(jnp.where(dists == min_d, iota_k, _K), axis=1)
    idx_ref[0, 0, :] = idx

    onehot = (iota_k == idx[:, None]).astype(jnp.float32)
    zq_hard = lax.dot_general(onehot, cb, (((1,), (0,)), ((), ())),
                              preferred_element_type=jnp.float32)
    zq_hard_ref[...] = zq_hard

    shifted = min_d - dists
    e = jnp.exp(shifted)
    w = e / jnp.sum(e, axis=1, keepdims=True)
    w_ref[...] = w
    zq_soft_ref[...] = lax.dot_general(w, cb, (((1,), (0,)), ((), ())),
                                       preferred_element_type=jnp.float32)

    part = jnp.sum(min_d).reshape(1, 1)

    @pl.when(i == 0)
    def _():
        loss_ref[...] = part

    @pl.when(i > 0)
    def _():
        loss_ref[...] += part


@jax.jit
def kernel(z_e, codebook):
    zq_soft, zq_hard, idx3, w, loss_sum = pl.pallas_call(
        _vq_block,
        grid=(_NB,),
        in_specs=[
            pl.BlockSpec((_RB, _D), lambda i: (i, 0)),
            pl.BlockSpec((_K, _D), lambda i: (0, 0)),
        ],
        out_specs=[
            pl.BlockSpec((_RB, _D), lambda i: (i, 0)),
            pl.BlockSpec((_RB, _D), lambda i: (i, 0)),
            pl.BlockSpec((1, 1, _RB), lambda i: (i, 0, 0)),
            pl.BlockSpec((_RB, _K), lambda i: (i, 0)),
            pl.BlockSpec((1, 1), lambda i: (0, 0)),
        ],
        out_shape=[
            jax.ShapeDtypeStruct((_B, _D), jnp.float32),
            jax.ShapeDtypeStruct((_B, _D), jnp.float32),
            jax.ShapeDtypeStruct((_NB, 1, _RB), jnp.int32),
            jax.ShapeDtypeStruct((_B, _K), jnp.float32),
            jax.ShapeDtypeStruct((1, 1), jnp.float32),
        ],
        compiler_params=pltpu.CompilerParams(
            dimension_semantics=("arbitrary",),
        ),
    )(z_e, codebook)
    indices = idx3.reshape(_B)
    mean_sq = loss_sum[0, 0] / (_B * _D)
    loss_vq = mean_sq + 0.5 * mean_sq
    return (zq_soft, zq_hard, indices, w, loss_vq)
